# baseline (speedup 1.0000x reference)
import functools

import numpy as np
import jax
import jax.numpy as jnp
from jax import lax
from jax.experimental import pallas as pl
from jax.experimental.pallas import tpu as pltpu
from jax.experimental.pallas import tpu_sc as plsc

L = 16
NC, NS = 2, 16
NW = NC * NS
GROUP_PAD = 4 * L

_GDN = lax.GatherDimensionNumbers(
    offset_dims=(), collapsed_slice_dims=(0,), start_index_map=(0,))


def _perm(v, idx):
    return lax.gather(v, idx[:, None], dimension_numbers=_GDN,
                      slice_sizes=(1,),
                      mode=lax.GatherScatterMode.PROMISE_IN_BOUNDS)


def _lane():
    return lax.iota(jnp.int32, L)


def _ce(k1, k2, payloads, j, k):
    ln = _lane()
    pidx = ln ^ j
    asc = (ln & k) == 0
    ks = asc == ((ln & j) == 0)
    p1 = _perm(k1, pidx)
    p2 = _perm(k2, pidx)
    less = (p1 < k1) | ((p1 == k1) & (p2 < k2))
    take = jnp.where(ks, less, ~less)
    o1 = jnp.where(take, p1, k1)
    o2 = jnp.where(take, p2, k2)
    pays = tuple(jnp.where(take, _perm(p, pidx), p) for p in payloads)
    return o1, o2, pays


def _sort16(k1, k2, payloads=()):
    for k in (2, 4, 8, 16):
        j = k // 2
        while j >= 1:
            k1, k2, payloads = _ce(k1, k2, payloads, j, k)
            j //= 2
    return k1, k2, payloads


def _merge16(k1, k2, payloads=()):
    for j in (8, 4, 2, 1):
        k1, k2, payloads = _ce(k1, k2, payloads, j, 32)
    return k1, k2, payloads


def _ce_explicit(k1, k2, payloads, pidx, ks):
    p1 = _perm(k1, pidx)
    p2 = _perm(k2, pidx)
    less = (p1 < k1) | ((p1 == k1) & (p2 < k2))
    lessi = less.astype(jnp.int32) ^ ks
    take = lessi != 0
    o1 = jnp.where(take, p1, k1)
    o2 = jnp.where(take, p2, k2)
    pays = tuple(jnp.where(take, _perm(p, pidx), p) for p in payloads)
    return o1, o2, pays


def _sort16_hw(k1, k2, payloads=()):
    k1, k2 = plsc.sort_key_val(k1, k2)
    ln = _lane()
    k1, k2, payloads = _ce_explicit(k1, k2, payloads, ln ^ 1, ln & 1)
    pidx = jnp.clip(((ln + 1) ^ 1) - 1, 0, L - 1)
    k1, k2, payloads = _ce_explicit(k1, k2, payloads, pidx, (ln & 1) ^ 1)
    return k1, k2, payloads


def _ce_pair(a1, a2, apays, b1, b2, bpays):
    bless = (b1 < a1) | ((b1 == a1) & (b2 < a2))
    lo1 = jnp.where(bless, b1, a1)
    lo2 = jnp.where(bless, b2, a2)
    hi1 = jnp.where(bless, a1, b1)
    hi2 = jnp.where(bless, a2, b2)
    lop = tuple(jnp.where(bless, bp, ap) for ap, bp in zip(apays, bpays))
    hip = tuple(jnp.where(bless, ap, bp) for ap, bp in zip(apays, bpays))
    return (lo1, lo2, lop), (hi1, hi2, hip)


def _rev(v):
    return lax.rev(v, dimensions=(0,))


def _merge_new(c0d, c0i, c1d, c1i, nd, ni):
    (s1d, s1i, _), _ = _ce_pair(c1d, c1i, (), _rev(nd), _rev(ni), ())
    s1d, s1i, _ = _merge16(s1d, s1i)
    (lod, loi, _), (hid, hii, _) = _ce_pair(c0d, c0i, (), _rev(s1d), _rev(s1i), ())
    lod, loi, _ = _merge16(lod, loi)
    hid, hii, _ = _merge16(hid, hii)
    return lod, loi, hid, hii


def _newton_sqrt(v):
    b = lax.bitcast_convert_type(v, jnp.int32)
    g = lax.bitcast_convert_type((b >> 1) + jnp.int32(0x1FBD1DF5), jnp.float32)
    for _ in range(4):
        g = 0.5 * (g + v / g)
    return g


def _vsplat(s):
    return jnp.full((L,), s, dtype=jnp.float32)


def _rerank(c0d, c0i, c1d, c1i, px0, py0, pz0, px1, py1, pz1):
    lane = _lane()
    mx = (_vsplat(jnp.sum(px0)) + _vsplat(jnp.sum(px1))) * (1.0 / 32.0)
    my = (_vsplat(jnp.sum(py0)) + _vsplat(jnp.sum(py1))) * (1.0 / 32.0)
    mz = (_vsplat(jnp.sum(pz0)) + _vsplat(jnp.sum(pz1))) * (1.0 / 32.0)
    ex0, ex1 = px0 - mx, px1 - mx
    ey0, ey1 = py0 - my, py1 - my
    ez0, ez1 = pz0 - mz, pz1 - mz
    ssd0 = (ex0 * ex0 + ey0 * ey0) + ez0 * ez0
    ssd1 = (ex1 * ex1 + ey1 * ey1) + ez1 * ez1
    vx = _vsplat(jnp.sum(ex0 * ex0)) + _vsplat(jnp.sum(ex1 * ex1))
    vy = _vsplat(jnp.sum(ey0 * ey0)) + _vsplat(jnp.sum(ey1 * ey1))
    vz = _vsplat(jnp.sum(ez0 * ez0)) + _vsplat(jnp.sum(ez1 * ez1))
    var_vec = jnp.where(lane == 0, vx, jnp.where(lane == 1, vy, vz))
    var_vec = var_vec / jnp.full((L,), 31.0, dtype=jnp.float32)
    std_vec = _newton_sqrt(var_vec)
    eps = jnp.full((L,), 1e-6, dtype=jnp.float32)
    s0 = _perm(std_vec, jnp.zeros((L,), jnp.int32)) + eps
    s1 = _perm(std_vec, jnp.full((L,), 1, jnp.int32)) + eps
    s2 = _perm(std_vec, jnp.full((L,), 2, jnp.int32)) + eps
    meanstd = ((s0 + s1) + s2) / jnp.full((L,), 3.0, dtype=jnp.float32)
    w0 = jnp.exp(-(ssd0 / meanstd))
    w1 = jnp.exp(-(ssd1 / meanstd))
    wd0 = c0d * w0
    wd1 = c1d * w1
    wd0, pos0, _ = _sort16_hw(wd0, lane)
    wd1, pos1, _ = _sort16_hw(wd1, lane)
    pos1 = pos1 + L
    (lwd, lpos, _), _ = _ce_pair(wd0, pos0, (), _rev(wd1), _rev(pos1), ())
    _, lpos, _ = _merge16(lwd, lpos)
    return lpos


def _knn_body(nq_per_worker, n_points, xt_hbm, qt_hbm, out_hbm,
              x_v, y_v, z_v, qx_v, qy_v, qz_v,
              pend_d, pend_i, cand_d, cand_i, out_v, smem):
    wid = lax.axis_index("s") * NC + lax.axis_index("c")
    n_per_batch = n_points
    workers_per_batch = n_per_batch // nq_per_worker
    b = wid // workers_per_batch
    qoff = (wid % workers_per_batch) * nq_per_worker
    plane = 3 * n_points
    base = b * plane
    pltpu.sync_copy(xt_hbm.at[pl.ds(base, n_points)], x_v.at[pl.ds(0, n_points)])
    pltpu.sync_copy(xt_hbm.at[pl.ds(base + n_points, n_points)],
                    y_v.at[pl.ds(0, n_points)])
    pltpu.sync_copy(xt_hbm.at[pl.ds(base + 2 * n_points, n_points)],
                    z_v.at[pl.ds(0, n_points)])
    pltpu.sync_copy(qt_hbm.at[pl.ds(base + qoff, nq_per_worker)], qx_v)
    pltpu.sync_copy(qt_hbm.at[pl.ds(base + n_points + qoff, nq_per_worker)], qy_v)
    pltpu.sync_copy(qt_hbm.at[pl.ds(base + 2 * n_points + qoff, nq_per_worker)], qz_v)

    lane = _lane()
    inf_v = jnp.full((L,), jnp.inf, dtype=jnp.float32)
    n_chunks = n_points // L
    for t in range(GROUP_PAD // L):
        x_v[pl.ds(n_points + t * L, L)] = inf_v
        y_v[pl.ds(n_points + t * L, L)] = inf_v
        z_v[pl.ds(n_points + t * L, L)] = inf_v

    def thr_ref_set(val):
        smem[1] = lax.bitcast_convert_type(val, jnp.int32)

    def thr_ref_get():
        return lax.bitcast_convert_type(smem[1], jnp.float32)

    def per_query(q, _):
        qblk = pl.multiple_of((q // L) * L, L)
        qsel = jnp.full((L,), q % L, dtype=jnp.int32)

        def bcast(ref):
            return _perm(ref[pl.ds(qblk, L)], qsel)

        qx = bcast(qx_v)
        qy = bcast(qy_v)
        qz = bcast(qz_v)
        smem[0] = 0

        def dist_at(off):
            xv = x_v[pl.ds(off, L)]
            yv = y_v[pl.ds(off, L)]
            zv = z_v[pl.ds(off, L)]
            dx = xv - qx
            dy = yv - qy
            dz = zv - qz
            return (dx * dx + dy * dy) + dz * dz

        d0 = dist_at(0)
        d1 = dist_at(L)
        i0 = lane
        i1 = lane + L
        d0, i0, _ = _sort16_hw(d0, i0)
        d1, i1, _ = _sort16_hw(d1, i1)
        (lod, loi, _), (hid, hii, _) = _ce_pair(d0, i0, (), _rev(d1), _rev(i1), ())
        lod, loi, _ = _merge16(lod, loi)
        hid, hii, _ = _merge16(hid, hii)
        cand_d[pl.ds(0, L)] = lod
        cand_i[pl.ds(0, L)] = loi
        cand_d[pl.ds(L, L)] = hid
        cand_i[pl.ds(L, L)] = hii
        thr0 = _perm(hid, jnp.full((L,), L - 1, jnp.int32))

        def flush_fn(thr):
            p_d = pend_d[pl.ds(0, L)]
            p_i = pend_i[pl.ds(0, L)]
            p_d, p_i, _ = _sort16_hw(p_d, p_i)
            c0d = cand_d[pl.ds(0, L)]
            c0i = cand_i[pl.ds(0, L)]
            c1d = cand_d[pl.ds(L, L)]
            c1i = cand_i[pl.ds(L, L)]
            c0d, c0i, c1d, c1i = _merge_new(c0d, c0i, c1d, c1i, p_d, p_i)
            cand_d[pl.ds(0, L)] = c0d
            cand_i[pl.ds(0, L)] = c0i
            cand_d[pl.ds(L, L)] = c1d
            cand_i[pl.ds(L, L)] = c1i
            pend_d[pl.ds(0, L)] = pend_d[pl.ds(L, L)]
            pend_i[pl.ds(0, L)] = pend_i[pl.ds(L, L)]
            pend_d[pl.ds(L, L)] = pend_d[pl.ds(2 * L, L)]
            pend_i[pl.ds(L, L)] = pend_i[pl.ds(2 * L, L)]
            return _perm(c1d, jnp.full((L,), L - 1, jnp.int32))

        def per_pair(j, thr):
            off = pl.multiple_of(2 * L + j * 2 * L, L)
            pcnt = smem[0]
            d_a = dist_at(off)
            m_a = d_a < thr
            pos_a = pcnt + jnp.cumsum(m_a.astype(jnp.int32)) - 1
            plsc.store_scatter(pend_d, [pos_a], d_a, mask=m_a)
            plsc.store_scatter(pend_i, [pos_a], jnp.int32(off) + lane,
                               mask=m_a)
            ca = plsc.all_reduce_population_count(m_a)
            d_b = dist_at(off + L)
            m_b = d_b < thr
            pos_b = (pcnt + ca) + jnp.cumsum(m_b.astype(jnp.int32)) - 1
            plsc.store_scatter(pend_d, [pos_b], d_b, mask=m_b)
            plsc.store_scatter(pend_i, [pos_b], jnp.int32(off) + L + lane,
                               mask=m_b)
            cb = plsc.all_reduce_population_count(m_b)
            newcnt = pcnt + (ca + cb)[0]
            nf = ((newcnt >= L).astype(jnp.int32)
                  + (newcnt >= 2 * L).astype(jnp.int32))
            smem[0] = newcnt - L * nf

            def flush2(t):
                t = flush_fn(t)
                return lax.cond(newcnt - L >= L, flush_fn, lambda a: a, t)

            return lax.cond(newcnt >= L, flush2, lambda a: a, thr)

        lax.fori_loop(0, (n_chunks - 2) // 2, per_pair, thr0, unroll=2)

        @pl.when(smem[0] > 0)
        def _final():
            fcnt = smem[0]
            p_d = pend_d[pl.ds(0, L)]
            p_i = pend_i[pl.ds(0, L)]
            mfin = lane < fcnt
            p_d = jnp.where(mfin, p_d, jnp.inf)
            p_i = jnp.where(mfin, p_i, jnp.int32(3 * n_points) + lane)
            p_d, p_i, _ = _sort16_hw(p_d, p_i)
            c0d = cand_d[pl.ds(0, L)]
            c0i = cand_i[pl.ds(0, L)]
            c1d = cand_d[pl.ds(L, L)]
            c1i = cand_i[pl.ds(L, L)]
            c0d, c0i, c1d, c1i = _merge_new(c0d, c0i, c1d, c1i, p_d, p_i)
            cand_d[pl.ds(0, L)] = c0d
            cand_i[pl.ds(0, L)] = c0i
            cand_d[pl.ds(L, L)] = c1d
            cand_i[pl.ds(L, L)] = c1i

        c0d = cand_d[pl.ds(0, L)]
        c0i = cand_i[pl.ds(0, L)]
        c1d = cand_d[pl.ds(L, L)]
        c1i = cand_i[pl.ds(L, L)]
        px0 = plsc.load_gather(x_v, [c0i])
        py0 = plsc.load_gather(y_v, [c0i])
        pz0 = plsc.load_gather(z_v, [c0i])
        px1 = plsc.load_gather(x_v, [c1i])
        py1 = plsc.load_gather(y_v, [c1i])
        pz1 = plsc.load_gather(z_v, [c1i])
        lpos = _rerank(c0d, c0i, c1d, c1i, px0, py0, pz0, px1, py1, pz1)
        lidx = plsc.load_gather(cand_i, [lpos])
        out_v[pl.ds(pl.multiple_of(q * L, L), L)] = lidx
        return 0

    lax.fori_loop(0, nq_per_worker, per_query, 0, unroll=False)
    pltpu.sync_copy(out_v, out_hbm.at[pl.ds(wid * nq_per_worker * L,
                                            nq_per_worker * L)])


def _knn_sc(xt, qt, b, n, m):
    nq_per_worker = (b * m) // NW
    body = functools.partial(_knn_body, nq_per_worker, n)
    mesh = plsc.VectorSubcoreMesh(core_axis_name="c", subcore_axis_name="s")
    f = pl.kernel(
        body,
        out_type=jax.ShapeDtypeStruct((b * m * 16,), jnp.int32),
        mesh=mesh,
        compiler_params=pltpu.CompilerParams(needs_layout_passes=False),
        scratch_types=[
            pltpu.VMEM((n + GROUP_PAD,), jnp.float32),
            pltpu.VMEM((n + GROUP_PAD,), jnp.float32),
            pltpu.VMEM((n + GROUP_PAD,), jnp.float32),
            pltpu.VMEM((nq_per_worker,), jnp.float32),
            pltpu.VMEM((nq_per_worker,), jnp.float32),
            pltpu.VMEM((nq_per_worker,), jnp.float32),
            pltpu.VMEM((3 * L,), jnp.float32),
            pltpu.VMEM((3 * L,), jnp.int32),
            pltpu.VMEM((2 * L,), jnp.float32),
            pltpu.VMEM((2 * L,), jnp.int32),
            pltpu.VMEM((nq_per_worker * L,), jnp.int32),
            pltpu.SMEM((8,), jnp.int32),
        ],
    )
    return f(xt, qt)


def kernel(xyz, new_xyz=None):
    if new_xyz is None:
        new_xyz = xyz
    b, n, _ = xyz.shape
    m = new_xyz.shape[1]
    xt = jnp.transpose(xyz, (0, 2, 1)).reshape(-1)
    qt = jnp.transpose(new_xyz, (0, 2, 1)).reshape(-1)
    out = _knn_sc(xt, qt, b, n, m)
    return out.reshape(b, m, 16)


# speedup vs baseline: 1.8695x; 1.8695x over previous
import functools

import numpy as np
import jax
import jax.numpy as jnp
from jax import lax
from jax.experimental import pallas as pl
from jax.experimental.pallas import tpu as pltpu
from jax.experimental.pallas import tpu_sc as plsc

L = 16
NC, NS = 2, 16
NW = NC * NS
GROUP_PAD = 4 * L

_GDN = lax.GatherDimensionNumbers(
    offset_dims=(), collapsed_slice_dims=(0,), start_index_map=(0,))


def _perm(v, idx):
    return lax.gather(v, idx[:, None], dimension_numbers=_GDN,
                      slice_sizes=(1,),
                      mode=lax.GatherScatterMode.PROMISE_IN_BOUNDS)


def _lane():
    return lax.iota(jnp.int32, L)


def _ce(k1, k2, payloads, j, k):
    ln = _lane()
    pidx = ln ^ j
    asc = (ln & k) == 0
    ks = asc == ((ln & j) == 0)
    p1 = _perm(k1, pidx)
    p2 = _perm(k2, pidx)
    less = (p1 < k1) | ((p1 == k1) & (p2 < k2))
    take = jnp.where(ks, less, ~less)
    o1 = jnp.where(take, p1, k1)
    o2 = jnp.where(take, p2, k2)
    pays = tuple(jnp.where(take, _perm(p, pidx), p) for p in payloads)
    return o1, o2, pays


def _sort16(k1, k2, payloads=()):
    for k in (2, 4, 8, 16):
        j = k // 2
        while j >= 1:
            k1, k2, payloads = _ce(k1, k2, payloads, j, k)
            j //= 2
    return k1, k2, payloads


def _merge16(k1, k2, payloads=()):
    for j in (8, 4, 2, 1):
        k1, k2, payloads = _ce(k1, k2, payloads, j, 32)
    return k1, k2, payloads


def _ce_explicit(k1, k2, payloads, pidx, ks):
    p1 = _perm(k1, pidx)
    p2 = _perm(k2, pidx)
    less = (p1 < k1) | ((p1 == k1) & (p2 < k2))
    lessi = less.astype(jnp.int32) ^ ks
    take = lessi != 0
    o1 = jnp.where(take, p1, k1)
    o2 = jnp.where(take, p2, k2)
    pays = tuple(jnp.where(take, _perm(p, pidx), p) for p in payloads)
    return o1, o2, pays


def _sort16_hw(k1, k2, payloads=()):
    k1, k2 = plsc.sort_key_val(k1, k2)
    ln = _lane()
    k1, k2, payloads = _ce_explicit(k1, k2, payloads, ln ^ 1, ln & 1)
    pidx = jnp.clip(((ln + 1) ^ 1) - 1, 0, L - 1)
    k1, k2, payloads = _ce_explicit(k1, k2, payloads, pidx, (ln & 1) ^ 1)
    return k1, k2, payloads


def _ce_pair(a1, a2, apays, b1, b2, bpays):
    bless = (b1 < a1) | ((b1 == a1) & (b2 < a2))
    lo1 = jnp.where(bless, b1, a1)
    lo2 = jnp.where(bless, b2, a2)
    hi1 = jnp.where(bless, a1, b1)
    hi2 = jnp.where(bless, a2, b2)
    lop = tuple(jnp.where(bless, bp, ap) for ap, bp in zip(apays, bpays))
    hip = tuple(jnp.where(bless, ap, bp) for ap, bp in zip(apays, bpays))
    return (lo1, lo2, lop), (hi1, hi2, hip)


def _rev(v):
    return lax.rev(v, dimensions=(0,))


def _merge_new(c0d, c0i, c1d, c1i, nd, ni):
    (s1d, s1i, _), _ = _ce_pair(c1d, c1i, (), _rev(nd), _rev(ni), ())
    s1d, s1i, _ = _merge16(s1d, s1i)
    (lod, loi, _), (hid, hii, _) = _ce_pair(c0d, c0i, (), _rev(s1d), _rev(s1i), ())
    lod, loi, _ = _merge16(lod, loi)
    hid, hii, _ = _merge16(hid, hii)
    return lod, loi, hid, hii


def _newton_sqrt(v):
    b = lax.bitcast_convert_type(v, jnp.int32)
    g = lax.bitcast_convert_type((b >> 1) + jnp.int32(0x1FBD1DF5), jnp.float32)
    for _ in range(4):
        g = 0.5 * (g + v / g)
    return g


def _vsplat(s):
    return jnp.full((L,), s, dtype=jnp.float32)


def _rerank(c0d, c0i, c1d, c1i, px0, py0, pz0, px1, py1, pz1):
    lane = _lane()
    mx = (_vsplat(jnp.sum(px0)) + _vsplat(jnp.sum(px1))) * (1.0 / 32.0)
    my = (_vsplat(jnp.sum(py0)) + _vsplat(jnp.sum(py1))) * (1.0 / 32.0)
    mz = (_vsplat(jnp.sum(pz0)) + _vsplat(jnp.sum(pz1))) * (1.0 / 32.0)
    ex0, ex1 = px0 - mx, px1 - mx
    ey0, ey1 = py0 - my, py1 - my
    ez0, ez1 = pz0 - mz, pz1 - mz
    ssd0 = (ex0 * ex0 + ey0 * ey0) + ez0 * ez0
    ssd1 = (ex1 * ex1 + ey1 * ey1) + ez1 * ez1
    vx = _vsplat(jnp.sum(ex0 * ex0)) + _vsplat(jnp.sum(ex1 * ex1))
    vy = _vsplat(jnp.sum(ey0 * ey0)) + _vsplat(jnp.sum(ey1 * ey1))
    vz = _vsplat(jnp.sum(ez0 * ez0)) + _vsplat(jnp.sum(ez1 * ez1))
    var_vec = jnp.where(lane == 0, vx, jnp.where(lane == 1, vy, vz))
    var_vec = var_vec / jnp.full((L,), 31.0, dtype=jnp.float32)
    std_vec = _newton_sqrt(var_vec)
    eps = jnp.full((L,), 1e-6, dtype=jnp.float32)
    s0 = _perm(std_vec, jnp.zeros((L,), jnp.int32)) + eps
    s1 = _perm(std_vec, jnp.full((L,), 1, jnp.int32)) + eps
    s2 = _perm(std_vec, jnp.full((L,), 2, jnp.int32)) + eps
    meanstd = ((s0 + s1) + s2) / jnp.full((L,), 3.0, dtype=jnp.float32)
    w0 = jnp.exp(-(ssd0 / meanstd))
    w1 = jnp.exp(-(ssd1 / meanstd))
    wd0 = c0d * w0
    wd1 = c1d * w1
    wd0, pos0, _ = _sort16_hw(wd0, lane)
    wd1, pos1, _ = _sort16_hw(wd1, lane)
    pos1 = pos1 + L
    (lwd, lpos, _), _ = _ce_pair(wd0, pos0, (), _rev(wd1), _rev(pos1), ())
    _, lpos, _ = _merge16(lwd, lpos)
    return lpos


def _knn_body(nq_per_worker, n_points, xt_hbm, qt_hbm, out_hbm,
              x_v, y_v, z_v, qx_v, qy_v, qz_v,
              pend_d, pend_i, cand_d, cand_i, out_v, smem):
    wid = lax.axis_index("s") * NC + lax.axis_index("c")
    n_per_batch = n_points
    workers_per_batch = n_per_batch // nq_per_worker
    b = wid // workers_per_batch
    qoff = (wid % workers_per_batch) * nq_per_worker
    plane = 3 * n_points
    base = b * plane
    pltpu.sync_copy(xt_hbm.at[pl.ds(base, n_points)], x_v.at[pl.ds(0, n_points)])
    pltpu.sync_copy(xt_hbm.at[pl.ds(base + n_points, n_points)],
                    y_v.at[pl.ds(0, n_points)])
    pltpu.sync_copy(xt_hbm.at[pl.ds(base + 2 * n_points, n_points)],
                    z_v.at[pl.ds(0, n_points)])
    pltpu.sync_copy(qt_hbm.at[pl.ds(base + qoff, nq_per_worker)], qx_v)
    pltpu.sync_copy(qt_hbm.at[pl.ds(base + n_points + qoff, nq_per_worker)], qy_v)
    pltpu.sync_copy(qt_hbm.at[pl.ds(base + 2 * n_points + qoff, nq_per_worker)], qz_v)

    lane = _lane()
    inf_v = jnp.full((L,), jnp.inf, dtype=jnp.float32)
    n_chunks = n_points // L
    for t in range(GROUP_PAD // L):
        x_v[pl.ds(n_points + t * L, L)] = inf_v
        y_v[pl.ds(n_points + t * L, L)] = inf_v
        z_v[pl.ds(n_points + t * L, L)] = inf_v

    def thr_ref_set(val):
        smem[1] = lax.bitcast_convert_type(val, jnp.int32)

    def thr_ref_get():
        return lax.bitcast_convert_type(smem[1], jnp.float32)

    def per_query(q, _):
        qblk = pl.multiple_of((q // L) * L, L)
        qsel = jnp.full((L,), q % L, dtype=jnp.int32)

        def bcast(ref):
            return _perm(ref[pl.ds(qblk, L)], qsel)

        qx = bcast(qx_v)
        qy = bcast(qy_v)
        qz = bcast(qz_v)
        smem[0] = 0

        def dist_at(off):
            xv = x_v[pl.ds(off, L)]
            yv = y_v[pl.ds(off, L)]
            zv = z_v[pl.ds(off, L)]
            dx = xv - qx
            dy = yv - qy
            dz = zv - qz
            return (dx * dx + dy * dy) + dz * dz

        d0 = dist_at(0)
        d1 = dist_at(L)
        i0 = lane
        i1 = lane + L
        d0, i0, _ = _sort16_hw(d0, i0)
        d1, i1, _ = _sort16_hw(d1, i1)
        (lod, loi, _), (hid, hii, _) = _ce_pair(d0, i0, (), _rev(d1), _rev(i1), ())
        lod, loi, _ = _merge16(lod, loi)
        hid, hii, _ = _merge16(hid, hii)
        thr0 = _perm(hid, jnp.full((L,), L - 1, jnp.int32))

        def flush_fn(carry):
            c0d, c0i, c1d, c1i, thr = carry
            p_d = pend_d[pl.ds(0, L)]
            p_i = pend_i[pl.ds(0, L)]
            p_d, p_i, _ = _sort16_hw(p_d, p_i)
            c0d, c0i, c1d, c1i = _merge_new(c0d, c0i, c1d, c1i, p_d, p_i)
            pend_d[pl.ds(0, L)] = pend_d[pl.ds(L, L)]
            pend_i[pl.ds(0, L)] = pend_i[pl.ds(L, L)]
            smem[0] = smem[0] - L
            thr = _perm(c1d, jnp.full((L,), L - 1, jnp.int32))
            return (c0d, c0i, c1d, c1i, thr)

        def per_chunk(j, carry):
            off = pl.multiple_of(2 * L + j * L, L)
            d = dist_at(off)
            m = d < carry[4]
            pcnt = smem[0]
            pos = pcnt + jnp.cumsum(m.astype(jnp.int32)) - 1
            idxv = jnp.int32(off) + lane
            plsc.store_scatter(pend_d, [pos], d, mask=m)
            plsc.store_scatter(pend_i, [pos], idxv, mask=m)
            cnt = plsc.all_reduce_population_count(m)[0]
            smem[0] = pcnt + cnt
            return lax.cond(pcnt + cnt >= L, flush_fn, lambda a: a, carry)

        carry = lax.fori_loop(0, n_chunks - 2, per_chunk,
                              (lod, loi, hid, hii, thr0), unroll=4)

        def final_fn(carry):
            c0d, c0i, c1d, c1i, thr = carry
            fcnt = smem[0]
            p_d = pend_d[pl.ds(0, L)]
            p_i = pend_i[pl.ds(0, L)]
            mfin = lane < fcnt
            p_d = jnp.where(mfin, p_d, jnp.inf)
            p_i = jnp.where(mfin, p_i, jnp.int32(3 * n_points) + lane)
            p_d, p_i, _ = _sort16_hw(p_d, p_i)
            c0d, c0i, c1d, c1i = _merge_new(c0d, c0i, c1d, c1i, p_d, p_i)
            return (c0d, c0i, c1d, c1i, thr)

        carry = lax.cond(smem[0] > 0, final_fn, lambda a: a, carry)
        c0d, c0i, c1d, c1i, _ = carry
        cand_i[pl.ds(0, L)] = c0i
        cand_i[pl.ds(L, L)] = c1i
        px0 = plsc.load_gather(x_v, [c0i])
        py0 = plsc.load_gather(y_v, [c0i])
        pz0 = plsc.load_gather(z_v, [c0i])
        px1 = plsc.load_gather(x_v, [c1i])
        py1 = plsc.load_gather(y_v, [c1i])
        pz1 = plsc.load_gather(z_v, [c1i])
        lpos = _rerank(c0d, c0i, c1d, c1i, px0, py0, pz0, px1, py1, pz1)
        lidx = plsc.load_gather(cand_i, [lpos])
        out_v[pl.ds(pl.multiple_of(q * L, L), L)] = lidx
        return 0

    lax.fori_loop(0, nq_per_worker, per_query, 0, unroll=False)
    pltpu.sync_copy(out_v, out_hbm.at[pl.ds(wid * nq_per_worker * L,
                                            nq_per_worker * L)])


def _knn_sc(xt, qt, b, n, m):
    nq_per_worker = (b * m) // NW
    body = functools.partial(_knn_body, nq_per_worker, n)
    mesh = plsc.VectorSubcoreMesh(core_axis_name="c", subcore_axis_name="s")
    f = pl.kernel(
        body,
        out_type=jax.ShapeDtypeStruct((b * m * 16,), jnp.int32),
        mesh=mesh,
        compiler_params=pltpu.CompilerParams(needs_layout_passes=False),
        scratch_types=[
            pltpu.VMEM((n + GROUP_PAD,), jnp.float32),
            pltpu.VMEM((n + GROUP_PAD,), jnp.float32),
            pltpu.VMEM((n + GROUP_PAD,), jnp.float32),
            pltpu.VMEM((nq_per_worker,), jnp.float32),
            pltpu.VMEM((nq_per_worker,), jnp.float32),
            pltpu.VMEM((nq_per_worker,), jnp.float32),
            pltpu.VMEM((3 * L,), jnp.float32),
            pltpu.VMEM((3 * L,), jnp.int32),
            pltpu.VMEM((2 * L,), jnp.float32),
            pltpu.VMEM((2 * L,), jnp.int32),
            pltpu.VMEM((nq_per_worker * L,), jnp.int32),
            pltpu.SMEM((8,), jnp.int32),
        ],
    )
    return f(xt, qt)


def kernel(xyz, new_xyz=None):
    if new_xyz is None:
        new_xyz = xyz
    b, n, _ = xyz.shape
    m = new_xyz.shape[1]
    xt = jnp.transpose(xyz, (0, 2, 1)).reshape(-1)
    qt = jnp.transpose(new_xyz, (0, 2, 1)).reshape(-1)
    out = _knn_sc(xt, qt, b, n, m)
    return out.reshape(b, m, 16)


# speedup vs baseline: 1.8922x; 1.0121x over previous
import functools

import numpy as np
import jax
import jax.numpy as jnp
from jax import lax
from jax.experimental import pallas as pl
from jax.experimental.pallas import tpu as pltpu
from jax.experimental.pallas import tpu_sc as plsc

L = 16
NC, NS = 2, 16
NW = NC * NS
GROUP_PAD = 4 * L

_GDN = lax.GatherDimensionNumbers(
    offset_dims=(), collapsed_slice_dims=(0,), start_index_map=(0,))


def _perm(v, idx):
    return lax.gather(v, idx[:, None], dimension_numbers=_GDN,
                      slice_sizes=(1,),
                      mode=lax.GatherScatterMode.PROMISE_IN_BOUNDS)


def _lane():
    return lax.iota(jnp.int32, L)


def _ce(k1, k2, payloads, j, k):
    ln = _lane()
    pidx = ln ^ j
    asc = (ln & k) == 0
    ks = asc == ((ln & j) == 0)
    p1 = _perm(k1, pidx)
    p2 = _perm(k2, pidx)
    less = (p1 < k1) | ((p1 == k1) & (p2 < k2))
    take = jnp.where(ks, less, ~less)
    o1 = jnp.where(take, p1, k1)
    o2 = jnp.where(take, p2, k2)
    pays = tuple(jnp.where(take, _perm(p, pidx), p) for p in payloads)
    return o1, o2, pays


def _sort16(k1, k2, payloads=()):
    for k in (2, 4, 8, 16):
        j = k // 2
        while j >= 1:
            k1, k2, payloads = _ce(k1, k2, payloads, j, k)
            j //= 2
    return k1, k2, payloads


def _merge16(k1, k2, payloads=()):
    for j in (8, 4, 2, 1):
        k1, k2, payloads = _ce(k1, k2, payloads, j, 32)
    return k1, k2, payloads


def _ce_explicit(k1, k2, payloads, pidx, ks):
    p1 = _perm(k1, pidx)
    p2 = _perm(k2, pidx)
    less = (p1 < k1) | ((p1 == k1) & (p2 < k2))
    lessi = less.astype(jnp.int32) ^ ks
    take = lessi != 0
    o1 = jnp.where(take, p1, k1)
    o2 = jnp.where(take, p2, k2)
    pays = tuple(jnp.where(take, _perm(p, pidx), p) for p in payloads)
    return o1, o2, pays


def _sort16_hw(k1, k2, payloads=()):
    k1, k2 = plsc.sort_key_val(k1, k2)
    ln = _lane()
    k1, k2, payloads = _ce_explicit(k1, k2, payloads, ln ^ 1, ln & 1)
    pidx = jnp.clip(((ln + 1) ^ 1) - 1, 0, L - 1)
    k1, k2, payloads = _ce_explicit(k1, k2, payloads, pidx, (ln & 1) ^ 1)
    return k1, k2, payloads


def _ce_pair(a1, a2, apays, b1, b2, bpays):
    bless = (b1 < a1) | ((b1 == a1) & (b2 < a2))
    lo1 = jnp.where(bless, b1, a1)
    lo2 = jnp.where(bless, b2, a2)
    hi1 = jnp.where(bless, a1, b1)
    hi2 = jnp.where(bless, a2, b2)
    lop = tuple(jnp.where(bless, bp, ap) for ap, bp in zip(apays, bpays))
    hip = tuple(jnp.where(bless, ap, bp) for ap, bp in zip(apays, bpays))
    return (lo1, lo2, lop), (hi1, hi2, hip)


def _rev(v):
    return lax.rev(v, dimensions=(0,))


def _merge_new(c0d, c0i, c1d, c1i, nd, ni):
    (s1d, s1i, _), _ = _ce_pair(c1d, c1i, (), _rev(nd), _rev(ni), ())
    s1d, s1i, _ = _merge16(s1d, s1i)
    (lod, loi, _), (hid, hii, _) = _ce_pair(c0d, c0i, (), _rev(s1d), _rev(s1i), ())
    lod, loi, _ = _merge16(lod, loi)
    hid, hii, _ = _merge16(hid, hii)
    return lod, loi, hid, hii


def _newton_sqrt(v):
    b = lax.bitcast_convert_type(v, jnp.int32)
    g = lax.bitcast_convert_type((b >> 1) + jnp.int32(0x1FBD1DF5), jnp.float32)
    for _ in range(4):
        g = 0.5 * (g + v / g)
    return g


def _vsplat(s):
    return jnp.full((L,), s, dtype=jnp.float32)


def _rerank(c0d, c0i, c1d, c1i, px0, py0, pz0, px1, py1, pz1):
    lane = _lane()
    mx = (_vsplat(jnp.sum(px0)) + _vsplat(jnp.sum(px1))) * (1.0 / 32.0)
    my = (_vsplat(jnp.sum(py0)) + _vsplat(jnp.sum(py1))) * (1.0 / 32.0)
    mz = (_vsplat(jnp.sum(pz0)) + _vsplat(jnp.sum(pz1))) * (1.0 / 32.0)
    ex0, ex1 = px0 - mx, px1 - mx
    ey0, ey1 = py0 - my, py1 - my
    ez0, ez1 = pz0 - mz, pz1 - mz
    ssd0 = (ex0 * ex0 + ey0 * ey0) + ez0 * ez0
    ssd1 = (ex1 * ex1 + ey1 * ey1) + ez1 * ez1
    vx = _vsplat(jnp.sum(ex0 * ex0)) + _vsplat(jnp.sum(ex1 * ex1))
    vy = _vsplat(jnp.sum(ey0 * ey0)) + _vsplat(jnp.sum(ey1 * ey1))
    vz = _vsplat(jnp.sum(ez0 * ez0)) + _vsplat(jnp.sum(ez1 * ez1))
    var_vec = jnp.where(lane == 0, vx, jnp.where(lane == 1, vy, vz))
    var_vec = var_vec / jnp.full((L,), 31.0, dtype=jnp.float32)
    std_vec = _newton_sqrt(var_vec)
    eps = jnp.full((L,), 1e-6, dtype=jnp.float32)
    s0 = _perm(std_vec, jnp.zeros((L,), jnp.int32)) + eps
    s1 = _perm(std_vec, jnp.full((L,), 1, jnp.int32)) + eps
    s2 = _perm(std_vec, jnp.full((L,), 2, jnp.int32)) + eps
    meanstd = ((s0 + s1) + s2) / jnp.full((L,), 3.0, dtype=jnp.float32)
    w0 = jnp.exp(-(ssd0 / meanstd))
    w1 = jnp.exp(-(ssd1 / meanstd))
    wd0 = c0d * w0
    wd1 = c1d * w1
    wd0, pos0, _ = _sort16_hw(wd0, lane)
    wd1, pos1, _ = _sort16_hw(wd1, lane)
    pos1 = pos1 + L
    (lwd, lpos, _), _ = _ce_pair(wd0, pos0, (), _rev(wd1), _rev(pos1), ())
    _, lpos, _ = _merge16(lwd, lpos)
    return lpos


def _knn_body(nq_per_worker, n_points, xt_hbm, qt_hbm, out_hbm,
              x_v, y_v, z_v, qx_v, qy_v, qz_v,
              pend_d, pend_i, cand_d, cand_i, out_v, smem):
    wid = lax.axis_index("s") * NC + lax.axis_index("c")
    n_per_batch = n_points
    workers_per_batch = n_per_batch // nq_per_worker
    b = wid // workers_per_batch
    qoff = (wid % workers_per_batch) * nq_per_worker
    plane = 3 * n_points
    base = b * plane
    pltpu.sync_copy(xt_hbm.at[pl.ds(base, n_points)], x_v.at[pl.ds(0, n_points)])
    pltpu.sync_copy(xt_hbm.at[pl.ds(base + n_points, n_points)],
                    y_v.at[pl.ds(0, n_points)])
    pltpu.sync_copy(xt_hbm.at[pl.ds(base + 2 * n_points, n_points)],
                    z_v.at[pl.ds(0, n_points)])
    pltpu.sync_copy(qt_hbm.at[pl.ds(base + qoff, nq_per_worker)], qx_v)
    pltpu.sync_copy(qt_hbm.at[pl.ds(base + n_points + qoff, nq_per_worker)], qy_v)
    pltpu.sync_copy(qt_hbm.at[pl.ds(base + 2 * n_points + qoff, nq_per_worker)], qz_v)

    lane = _lane()
    inf_v = jnp.full((L,), jnp.inf, dtype=jnp.float32)
    n_chunks = n_points // L
    for t in range(GROUP_PAD // L):
        x_v[pl.ds(n_points + t * L, L)] = inf_v
        y_v[pl.ds(n_points + t * L, L)] = inf_v
        z_v[pl.ds(n_points + t * L, L)] = inf_v

    def thr_ref_set(val):
        smem[1] = lax.bitcast_convert_type(val, jnp.int32)

    def thr_ref_get():
        return lax.bitcast_convert_type(smem[1], jnp.float32)

    def per_query(q, _):
        qblk = pl.multiple_of((q // L) * L, L)
        qsel = jnp.full((L,), q % L, dtype=jnp.int32)

        def bcast(ref):
            return _perm(ref[pl.ds(qblk, L)], qsel)

        qx = bcast(qx_v)
        qy = bcast(qy_v)
        qz = bcast(qz_v)
        smem[0] = 0

        def dist_at(off):
            xv = x_v[pl.ds(off, L)]
            yv = y_v[pl.ds(off, L)]
            zv = z_v[pl.ds(off, L)]
            dx = xv - qx
            dy = yv - qy
            dz = zv - qz
            return (dx * dx + dy * dy) + dz * dz

        d0 = dist_at(0)
        d1 = dist_at(L)
        i0 = lane
        i1 = lane + L
        d0, i0, _ = _sort16_hw(d0, i0)
        d1, i1, _ = _sort16_hw(d1, i1)
        (lod, loi, _), (hid, hii, _) = _ce_pair(d0, i0, (), _rev(d1), _rev(i1), ())
        lod, loi, _ = _merge16(lod, loi)
        hid, hii, _ = _merge16(hid, hii)
        thr0 = _perm(hid, jnp.full((L,), L - 1, jnp.int32))

        def flush_fn(carry):
            c0d, c0i, c1d, c1i, thr = carry
            p_d = pend_d[pl.ds(0, L)]
            p_i = pend_i[pl.ds(0, L)]
            p_d, p_i, _ = _sort16_hw(p_d, p_i)
            c0d, c0i, c1d, c1i = _merge_new(c0d, c0i, c1d, c1i, p_d, p_i)
            pend_d[pl.ds(0, L)] = pend_d[pl.ds(L, L)]
            pend_i[pl.ds(0, L)] = pend_i[pl.ds(L, L)]
            smem[0] = smem[0] - L
            thr = _perm(c1d, jnp.full((L,), L - 1, jnp.int32))
            return (c0d, c0i, c1d, c1i, thr)

        def per_chunk(j, carry):
            off = pl.multiple_of(2 * L + j * L, L)
            d = dist_at(off)
            m = d < carry[4]
            pcnt = smem[0]
            pos = pcnt + jnp.cumsum(m.astype(jnp.int32)) - 1
            idxv = jnp.int32(off) + lane
            plsc.store_scatter(pend_d, [pos], d, mask=m)
            plsc.store_scatter(pend_i, [pos], idxv, mask=m)
            cnt = plsc.all_reduce_population_count(m)[0]
            smem[0] = pcnt + cnt
            return lax.cond(pcnt + cnt >= L, flush_fn, lambda a: a, carry)

        carry = lax.fori_loop(0, n_chunks - 2, per_chunk,
                              (lod, loi, hid, hii, thr0), unroll=6)

        def final_fn(carry):
            c0d, c0i, c1d, c1i, thr = carry
            fcnt = smem[0]
            p_d = pend_d[pl.ds(0, L)]
            p_i = pend_i[pl.ds(0, L)]
            mfin = lane < fcnt
            p_d = jnp.where(mfin, p_d, jnp.inf)
            p_i = jnp.where(mfin, p_i, jnp.int32(3 * n_points) + lane)
            p_d, p_i, _ = _sort16_hw(p_d, p_i)
            c0d, c0i, c1d, c1i = _merge_new(c0d, c0i, c1d, c1i, p_d, p_i)
            return (c0d, c0i, c1d, c1i, thr)

        carry = lax.cond(smem[0] > 0, final_fn, lambda a: a, carry)
        c0d, c0i, c1d, c1i, _ = carry
        cand_i[pl.ds(0, L)] = c0i
        cand_i[pl.ds(L, L)] = c1i
        px0 = plsc.load_gather(x_v, [c0i])
        py0 = plsc.load_gather(y_v, [c0i])
        pz0 = plsc.load_gather(z_v, [c0i])
        px1 = plsc.load_gather(x_v, [c1i])
        py1 = plsc.load_gather(y_v, [c1i])
        pz1 = plsc.load_gather(z_v, [c1i])
        lpos = _rerank(c0d, c0i, c1d, c1i, px0, py0, pz0, px1, py1, pz1)
        lidx = plsc.load_gather(cand_i, [lpos])
        out_v[pl.ds(pl.multiple_of(q * L, L), L)] = lidx
        return 0

    lax.fori_loop(0, nq_per_worker, per_query, 0, unroll=False)
    pltpu.sync_copy(out_v, out_hbm.at[pl.ds(wid * nq_per_worker * L,
                                            nq_per_worker * L)])


def _knn_sc(xt, qt, b, n, m):
    nq_per_worker = (b * m) // NW
    body = functools.partial(_knn_body, nq_per_worker, n)
    mesh = plsc.VectorSubcoreMesh(core_axis_name="c", subcore_axis_name="s")
    f = pl.kernel(
        body,
        out_type=jax.ShapeDtypeStruct((b * m * 16,), jnp.int32),
        mesh=mesh,
        compiler_params=pltpu.CompilerParams(needs_layout_passes=False),
        scratch_types=[
            pltpu.VMEM((n + GROUP_PAD,), jnp.float32),
            pltpu.VMEM((n + GROUP_PAD,), jnp.float32),
            pltpu.VMEM((n + GROUP_PAD,), jnp.float32),
            pltpu.VMEM((nq_per_worker,), jnp.float32),
            pltpu.VMEM((nq_per_worker,), jnp.float32),
            pltpu.VMEM((nq_per_worker,), jnp.float32),
            pltpu.VMEM((3 * L,), jnp.float32),
            pltpu.VMEM((3 * L,), jnp.int32),
            pltpu.VMEM((2 * L,), jnp.float32),
            pltpu.VMEM((2 * L,), jnp.int32),
            pltpu.VMEM((nq_per_worker * L,), jnp.int32),
            pltpu.SMEM((8,), jnp.int32),
        ],
    )
    return f(xt, qt)


def kernel(xyz, new_xyz=None):
    if new_xyz is None:
        new_xyz = xyz
    b, n, _ = xyz.shape
    m = new_xyz.shape[1]
    xt = jnp.transpose(xyz, (0, 2, 1)).reshape(-1)
    qt = jnp.transpose(new_xyz, (0, 2, 1)).reshape(-1)
    out = _knn_sc(xt, qt, b, n, m)
    return out.reshape(b, m, 16)
